# Initial kernel scaffold; baseline (speedup 1.0000x reference)
import jax
import jax.numpy as jnp
from jax.experimental import pallas as pl


def kernel(input_x, table):
    raise NotImplementedError("write your pallas kernel here")


# speedup vs baseline: 5.5781x; 5.5781x over previous
import functools

import jax
import jax.numpy as jnp
from jax import lax
from jax.experimental import pallas as pl
from jax.experimental.pallas import tpu as pltpu
from jax.experimental.pallas import tpu_sc as plsc

B = 16384
L = 200
E = 32
V = 257

NC = 2
NS = 16
NW = NC * NS
BPW = B // NW
NB = 4
NIT = BPW // NB
LCH = 13
ROW = E * L


def _emb_body(tab_hbm, x_hbm, out_hbm, tab_v, idx_v, out_v):
    cid = lax.axis_index("c")
    sid = lax.axis_index("s")
    wid = sid * NC + cid

    pltpu.sync_copy(tab_hbm, tab_v)
    idx_v[pl.ds(NB * L, 16)] = jnp.zeros((16,), jnp.int32)

    lanes = lax.broadcasted_iota(jnp.int32, (16,), 0)
    m8 = lanes < 8

    def it_body(it, _):
        b0 = wid * BPW + it * NB
        pltpu.sync_copy(x_hbm.at[pl.ds(b0 * L, NB * L)],
                        idx_v.at[pl.ds(0, NB * L)])
        for r in range(NB):
            def chunk(c, _, r=r):
                iv = idx_v[pl.ds(r * L + c * 16, 16)]
                for e in range(E):
                    g = plsc.load_gather(tab_v, [iv + e * V])
                    out_v[pl.ds(r * ROW + e * L + c * 16, 16)] = g
                return 0
            lax.fori_loop(0, LCH - 1, chunk, 0)
            iv = idx_v[pl.ds(r * L + 192, 16)]
            for e in range(E):
                g = plsc.load_gather(tab_v, [iv + e * V])
                plsc.store_scatter(
                    out_v, [lanes + (r * ROW + e * L + 192)], g, mask=m8)
        pltpu.sync_copy(out_v.at[pl.ds(0, NB * ROW)],
                        out_hbm.at[pl.ds(b0 * ROW, NB * ROW)])
        return 0

    lax.fori_loop(0, NIT, it_body, 0)


@functools.partial(jax.jit, static_argnames=())
def kernel(input_x, table):
    x = input_x.reshape(B, L).astype(jnp.int32).reshape(-1)
    tab_t = jnp.transpose(table).reshape(-1)

    run = pl.kernel(
        _emb_body,
        out_type=jax.ShapeDtypeStruct((B * ROW,), jnp.float32),
        mesh=plsc.VectorSubcoreMesh(
            core_axis_name="c", subcore_axis_name="s",
            num_cores=NC, num_subcores=NS),
        scratch_types=[
            pltpu.VMEM((E * V,), jnp.float32),
            pltpu.VMEM((NB * L + 16,), jnp.int32),
            pltpu.VMEM((NB * ROW + 16,), jnp.float32),
        ],
        compiler_params=pltpu.CompilerParams(needs_layout_passes=False),
    )
    out = run(tab_t, x)
    return out.reshape(B, E, L)


# speedup vs baseline: 6.1285x; 1.0987x over previous
import functools

import jax
import jax.numpy as jnp
from jax import lax
from jax.experimental import pallas as pl
from jax.experimental.pallas import tpu as pltpu
from jax.experimental.pallas import tpu_sc as plsc

B = 16384
L = 200
E = 32
V = 257

NC = 2
NS = 16
NW = NC * NS
BPW = B // NW
NB = 8
NIT = BPW // NB
LCH = 13
ROW = E * L


def _emb_body(tab_hbm, x_hbm, out_hbm, tab_v,
              idx_a, idx_b, out_a, out_b,
              isem_a, isem_b, osem_a, osem_b):
    cid = lax.axis_index("c")
    sid = lax.axis_index("s")
    wid = sid * NC + cid
    base_b = wid * BPW

    pltpu.sync_copy(tab_hbm, tab_v)
    for v in (idx_a, idx_b):
        v[pl.ds(NB * L, 16)] = jnp.zeros((16,), jnp.int32)

    lanes = lax.broadcasted_iota(jnp.int32, (16,), 0)
    m8 = lanes < 8

    def idx_copy(i, v, sem):
        b0 = base_b + i * NB
        return pltpu.make_async_copy(
            x_hbm.at[pl.ds(b0 * L, NB * L)], v.at[pl.ds(0, NB * L)], sem)

    def out_copy(i, v, sem):
        b0 = base_b + i * NB
        return pltpu.make_async_copy(
            v.at[pl.ds(0, NB * ROW)], out_hbm.at[pl.ds(b0 * ROW, NB * ROW)],
            sem)

    def compute(idxv, outv):
        def row(r, _):
            def chunk(c, _):
                iv = idxv[pl.ds(r * L + c * 16, 16)]
                for e in range(E):
                    g = plsc.load_gather(tab_v, [iv + e * V])
                    outv[pl.ds(r * ROW + e * L + c * 16, 16)] = g
                return 0
            lax.fori_loop(0, LCH - 1, chunk, 0)
            iv = idxv[pl.ds(r * L + 192, 16)]
            for e in range(E):
                g = plsc.load_gather(tab_v, [iv + e * V])
                plsc.store_scatter(
                    outv, [lanes + (r * ROW + e * L + 192)], g, mask=m8)
            return 0
        lax.fori_loop(0, NB, row, 0)

    bufs = ((idx_a, out_a, isem_a, osem_a), (idx_b, out_b, isem_b, osem_b))

    idx_copy(0, idx_a, isem_a).start()
    idx_copy(1, idx_b, isem_b).start()
    for i in (0, 1):
        idxv, outv, isem, osem = bufs[i]
        idx_copy(i, idxv, isem).wait()
        compute(idxv, outv)
        out_copy(i, outv, osem).start()
        idx_copy(i + 2, idxv, isem).start()

    def steady(h, _):
        for b in range(2):
            i = 2 * h + b
            idxv, outv, isem, osem = bufs[b]
            idx_copy(i, idxv, isem).wait()
            out_copy(i - 2, outv, osem).wait()
            compute(idxv, outv)
            out_copy(i, outv, osem).start()
            idx_copy(lax.rem(i + 2, NIT), idxv, isem).start()
        return 0
    lax.fori_loop(1, NIT // 2, steady, 0)

    for b in range(2):
        idxv, outv, isem, osem = bufs[b]
        out_copy(NIT - 2 + b, outv, osem).wait()
        idx_copy(b, idxv, isem).wait()


@functools.partial(jax.jit, static_argnames=())
def kernel(input_x, table):
    x = input_x.reshape(B, L).astype(jnp.int32).reshape(-1)
    tab_t = jnp.transpose(table).reshape(-1)

    run = pl.kernel(
        _emb_body,
        out_type=jax.ShapeDtypeStruct((B * ROW,), jnp.float32),
        mesh=plsc.VectorSubcoreMesh(
            core_axis_name="c", subcore_axis_name="s",
            num_cores=NC, num_subcores=NS),
        scratch_types=[
            pltpu.VMEM((E * V,), jnp.float32),
            pltpu.VMEM((NB * L + 16,), jnp.int32),
            pltpu.VMEM((NB * L + 16,), jnp.int32),
            pltpu.VMEM((NB * ROW + 16,), jnp.float32),
            pltpu.VMEM((NB * ROW + 16,), jnp.float32),
            pltpu.SemaphoreType.DMA,
            pltpu.SemaphoreType.DMA,
            pltpu.SemaphoreType.DMA,
            pltpu.SemaphoreType.DMA,
        ],
        compiler_params=pltpu.CompilerParams(needs_layout_passes=False),
    )
    out = run(tab_t, x)
    return out.reshape(B, E, L)


# speedup vs baseline: 8.8462x; 1.4435x over previous
import functools

import jax
import jax.numpy as jnp
from jax import lax
from jax.experimental import pallas as pl
from jax.experimental.pallas import tpu as pltpu
from jax.experimental.pallas import tpu_sc as plsc

B = 16384
L = 200
E = 32
V = 257

NC = 2
NS = 16
NW = NC * NS
BPW = B // NW
NB = 8
NIT = BPW // NB
LCH = 13
ROW = E * L


def _emb_body(tab_hbm, x_hbm, out_hbm, tab_v,
              idx_a, idx_b, out_a, out_b,
              isem_a, isem_b, osem_a, osem_b):
    cid = lax.axis_index("c")
    sid = lax.axis_index("s")
    wid = sid * NC + cid
    base_b = wid * BPW

    pltpu.sync_copy(tab_hbm, tab_v)
    for v in (idx_a, idx_b):
        v[pl.ds(NB * L, 16)] = jnp.zeros((16,), jnp.int32)

    lanes = lax.broadcasted_iota(jnp.int32, (16,), 0)
    m8 = lanes < 8

    def idx_copy(i, v, sem):
        b0 = base_b + i * NB
        return pltpu.make_async_copy(
            x_hbm.at[pl.ds(b0 * L, NB * L)], v.at[pl.ds(0, NB * L)], sem)

    def out_copy(i, v, sem):
        b0 = base_b + i * NB
        return pltpu.make_async_copy(
            v.at[pl.ds(0, NB * ROW)], out_hbm.at[pl.ds(b0 * ROW, NB * ROW)],
            sem)

    def compute(idxv, outv):
        def row(r, _):
            def chunk(c, _):
                iv = idxv[pl.ds(r * L + c * 16, 16)]
                gs = [plsc.load_gather(tab_v, [iv + e * V])
                      for e in range(E)]
                for e in range(E):
                    outv[pl.ds(r * ROW + e * L + c * 16, 16)] = gs[e]
                return 0
            lax.fori_loop(0, LCH - 1, chunk, 0)
            iv = idxv[pl.ds(r * L + 192, 16)]
            gs = [plsc.load_gather(tab_v, [iv + e * V]) for e in range(E)]
            for e in range(E):
                plsc.store_scatter(
                    outv, [lanes + (r * ROW + e * L + 192)], gs[e], mask=m8)
            return 0
        lax.fori_loop(0, NB, row, 0)

    bufs = ((idx_a, out_a, isem_a, osem_a), (idx_b, out_b, isem_b, osem_b))

    idx_copy(0, idx_a, isem_a).start()
    idx_copy(1, idx_b, isem_b).start()
    for i in (0, 1):
        idxv, outv, isem, osem = bufs[i]
        idx_copy(i, idxv, isem).wait()
        compute(idxv, outv)
        out_copy(i, outv, osem).start()
        idx_copy(i + 2, idxv, isem).start()

    def steady(h, _):
        for b in range(2):
            i = 2 * h + b
            idxv, outv, isem, osem = bufs[b]
            idx_copy(i, idxv, isem).wait()
            out_copy(i - 2, outv, osem).wait()
            compute(idxv, outv)
            out_copy(i, outv, osem).start()
            idx_copy(lax.rem(i + 2, NIT), idxv, isem).start()
        return 0
    lax.fori_loop(1, NIT // 2, steady, 0)

    for b in range(2):
        idxv, outv, isem, osem = bufs[b]
        out_copy(NIT - 2 + b, outv, osem).wait()
        idx_copy(b, idxv, isem).wait()


@functools.partial(jax.jit, static_argnames=())
def kernel(input_x, table):
    x = input_x.reshape(B, L).astype(jnp.int32).reshape(-1)
    tab_t = jnp.transpose(table).reshape(-1)

    run = pl.kernel(
        _emb_body,
        out_type=jax.ShapeDtypeStruct((B * ROW,), jnp.float32),
        mesh=plsc.VectorSubcoreMesh(
            core_axis_name="c", subcore_axis_name="s",
            num_cores=NC, num_subcores=NS),
        scratch_types=[
            pltpu.VMEM((E * V,), jnp.float32),
            pltpu.VMEM((NB * L + 16,), jnp.int32),
            pltpu.VMEM((NB * L + 16,), jnp.int32),
            pltpu.VMEM((NB * ROW + 16,), jnp.float32),
            pltpu.VMEM((NB * ROW + 16,), jnp.float32),
            pltpu.SemaphoreType.DMA,
            pltpu.SemaphoreType.DMA,
            pltpu.SemaphoreType.DMA,
            pltpu.SemaphoreType.DMA,
        ],
        compiler_params=pltpu.CompilerParams(needs_layout_passes=False),
    )
    out = run(tab_t, x)
    return out.reshape(B, E, L)


# speedup vs baseline: 16.6397x; 1.8810x over previous
import functools

import jax
import jax.numpy as jnp
from jax import lax
from jax.experimental import pallas as pl
from jax.experimental.pallas import tpu as pltpu
from jax.experimental.pallas import tpu_sc as plsc

B = 16384
L = 200
E = 32
V = 257

NC = 2
NS = 16
NW = NC * NS
BPW = B // NW
NB = 4
NIT = BPW // NB
LCH = 13
ROW = E * L


def _emb_body(tab_hbm, x_hbm, out_hbm, tab_v,
              idx_a, idx_b, out_a, out_b,
              isem_a, isem_b, osem_a, osem_b):
    cid = lax.axis_index("c")
    sid = lax.axis_index("s")
    wid = sid * NC + cid
    base_b = wid * BPW

    pltpu.sync_copy(tab_hbm, tab_v)
    for v in (idx_a, idx_b):
        v[pl.ds(NB * L, 16)] = jnp.zeros((16,), jnp.int32)

    lanes = lax.broadcasted_iota(jnp.int32, (16,), 0)
    m8 = lanes < 8

    def idx_copy(i, v, sem):
        b0 = base_b + i * NB
        return pltpu.make_async_copy(
            x_hbm.at[pl.ds(b0 * L, NB * L)], v.at[pl.ds(0, NB * L)], sem)

    def out_copy(i, v, sem):
        b0 = base_b + i * NB
        return pltpu.make_async_copy(
            v.at[pl.ds(0, NB), :, :], out_hbm.at[pl.ds(b0, NB), :, :], sem)

    def compute(idxv, outv):
        def row(r, _):
            def chunk(c, _):
                iv = idxv[pl.ds(r * L + c * 16, 16)]
                gs = [plsc.load_gather(tab_v, [iv + e * V])
                      for e in range(E)]
                for e in range(E):
                    outv[r, e, pl.ds(c * 16, 16)] = gs[e]
                return 0
            lax.fori_loop(0, LCH - 1, chunk, 0)
            iv = idxv[pl.ds(r * L + 192, 16)]
            gs = [plsc.load_gather(tab_v, [iv + e * V]) for e in range(E)]
            rr = jnp.full((16,), r, jnp.int32)
            for e in range(E):
                plsc.store_scatter(
                    outv, [rr, jnp.full((16,), e, jnp.int32), lanes + 192],
                    gs[e], mask=m8)
            return 0
        lax.fori_loop(0, NB, row, 0)

    bufs = ((idx_a, out_a, isem_a, osem_a), (idx_b, out_b, isem_b, osem_b))

    idx_copy(0, idx_a, isem_a).start()
    idx_copy(1, idx_b, isem_b).start()
    for i in (0, 1):
        idxv, outv, isem, osem = bufs[i]
        idx_copy(i, idxv, isem).wait()
        compute(idxv, outv)
        out_copy(i, outv, osem).start()
        idx_copy(i + 2, idxv, isem).start()

    def steady(h, _):
        for b in range(2):
            i = 2 * h + b
            idxv, outv, isem, osem = bufs[b]
            idx_copy(i, idxv, isem).wait()
            out_copy(i - 2, outv, osem).wait()
            compute(idxv, outv)
            out_copy(i, outv, osem).start()
            idx_copy(lax.rem(i + 2, NIT), idxv, isem).start()
        return 0
    lax.fori_loop(1, NIT // 2, steady, 0)

    for b in range(2):
        idxv, outv, isem, osem = bufs[b]
        out_copy(NIT - 2 + b, outv, osem).wait()
        idx_copy(b, idxv, isem).wait()


@functools.partial(jax.jit, static_argnames=())
def kernel(input_x, table):
    x = input_x.reshape(B, L).astype(jnp.int32).reshape(-1)
    tab_t = jnp.transpose(table).reshape(-1)

    run = pl.kernel(
        _emb_body,
        out_type=jax.ShapeDtypeStruct((B, E, L), jnp.float32),
        mesh=plsc.VectorSubcoreMesh(
            core_axis_name="c", subcore_axis_name="s",
            num_cores=NC, num_subcores=NS),
        scratch_types=[
            pltpu.VMEM((E * V,), jnp.float32),
            pltpu.VMEM((NB * L + 16,), jnp.int32),
            pltpu.VMEM((NB * L + 16,), jnp.int32),
            pltpu.VMEM((NB, E, L), jnp.float32),
            pltpu.VMEM((NB, E, L), jnp.float32),
            pltpu.SemaphoreType.DMA,
            pltpu.SemaphoreType.DMA,
            pltpu.SemaphoreType.DMA,
            pltpu.SemaphoreType.DMA,
        ],
        compiler_params=pltpu.CompilerParams(needs_layout_passes=False),
    )
    return run(tab_t, x)


# speedup vs baseline: 32.9753x; 1.9817x over previous
import functools

import jax
import jax.numpy as jnp
from jax import lax
from jax.experimental import pallas as pl
from jax.experimental.pallas import tpu as pltpu
from jax.experimental.pallas import tpu_sc as plsc

B = 16384
L = 200
E = 32
V = 257

NC = 2
NS = 16
NW = NC * NS
P = B * L
NEG = 4
NUG = NW // NEG
EPG = E // NEG
UW = 1024
UNITS = P // UW
UPG = UNITS // NUG
SU = 4
SW = SU * UW
NIT = UPG // SU
NV = SW // 16


def _emb_body(tab_hbm, x_hbm, out_hbm, tab_v,
              x_a, x_b, out_a, out_b,
              isem_a, isem_b, osem_a, osem_b):
    cid = lax.axis_index("c")
    sid = lax.axis_index("s")
    wid = sid * NC + cid
    eg = lax.rem(wid, NEG)
    ug = lax.div(wid, NEG)
    e0 = eg * EPG
    ebias = e0 * V

    pltpu.sync_copy(tab_hbm, tab_v)

    def x_copy(i, v, sem):
        off = (ug * UPG + i * SU) * UW
        return pltpu.make_async_copy(
            x_hbm.at[pl.ds(off, SW)], v, sem)

    def out_copies(i, v, sem):
        u0 = ug * UPG + i * SU
        return [pltpu.make_async_copy(
                    v.at[pl.ds(j * SW, SW)],
                    out_hbm.at[pl.ds((e0 + j) * P + u0 * UW, SW)], sem)
                for j in range(EPG)]

    def out_start(i, v, sem):
        for c in out_copies(i, v, sem):
            c.start()

    def out_wait(i, v, sem):
        for c in out_copies(i, v, sem):
            c.wait()

    def compute(xv, ov):
        def vec(vi, _):
            iv = xv[pl.ds(vi * 16, 16)] + ebias
            gs = [plsc.load_gather(tab_v, [iv + j * V])
                  for j in range(EPG)]
            for j in range(EPG):
                ov[pl.ds(j * SW + vi * 16, 16)] = gs[j]
            return 0
        lax.fori_loop(0, NV, vec, 0)

    bufs = ((x_a, out_a, isem_a, osem_a), (x_b, out_b, isem_b, osem_b))

    x_copy(0, x_a, isem_a).start()
    x_copy(1, x_b, isem_b).start()
    for i in (0, 1):
        xv, ov, isem, osem = bufs[i]
        x_copy(i, xv, isem).wait()
        compute(xv, ov)
        out_start(i, ov, osem)
        x_copy(i + 2, xv, isem).start()

    def steady(h, _):
        for b in range(2):
            i = 2 * h + b
            xv, ov, isem, osem = bufs[b]
            x_copy(i, xv, isem).wait()
            out_wait(i - 2, ov, osem)
            compute(xv, ov)
            out_start(i, ov, osem)
            x_copy(lax.rem(i + 2, NIT), xv, isem).start()
        return 0
    lax.fori_loop(1, NIT // 2, steady, 0)

    for b in range(2):
        xv, ov, isem, osem = bufs[b]
        out_wait(NIT - 2 + b, ov, osem)
        x_copy(b, xv, isem).wait()


@functools.partial(jax.jit, static_argnames=())
def kernel(input_x, table):
    xs = jnp.squeeze(input_x, 1).astype(jnp.int32)
    x_phys = xs.reshape(128, 128, 25, 8).transpose(2, 0, 3, 1).reshape(-1)
    tab_t = jnp.transpose(table).reshape(-1)

    run = pl.kernel(
        _emb_body,
        out_type=jax.ShapeDtypeStruct((E * P,), jnp.float32),
        mesh=plsc.VectorSubcoreMesh(
            core_axis_name="c", subcore_axis_name="s",
            num_cores=NC, num_subcores=NS),
        scratch_types=[
            pltpu.VMEM((E * V,), jnp.float32),
            pltpu.VMEM((SW,), jnp.int32),
            pltpu.VMEM((SW,), jnp.int32),
            pltpu.VMEM((EPG * SW,), jnp.float32),
            pltpu.VMEM((EPG * SW,), jnp.float32),
            pltpu.SemaphoreType.DMA,
            pltpu.SemaphoreType.DMA,
            pltpu.SemaphoreType.DMA,
            pltpu.SemaphoreType.DMA,
        ],
        compiler_params=pltpu.CompilerParams(needs_layout_passes=False),
    )
    out_phys = run(tab_t, x_phys)
    return (out_phys.reshape(E, 25, 128, 8, 128)
            .transpose(2, 4, 0, 1, 3).reshape(B, E, L))


# speedup vs baseline: 40.0261x; 1.2138x over previous
import functools

import jax
import jax.numpy as jnp
from jax import lax
from jax.experimental import pallas as pl
from jax.experimental.pallas import tpu as pltpu
from jax.experimental.pallas import tpu_sc as plsc

B = 16384
L = 200
E = 32
V = 257

NC = 2
NS = 16
NW = NC * NS
P = B * L
NEG = 4
NUG = NW // NEG
EPG = E // NEG
UW = 1024
UNITS = P // UW
UPG = UNITS // NUG
SU = 4
SW = SU * UW
NIT = UPG // SU
NV = SW // 16


def _emb_body(tab_hbm, x_hbm, out_hbm, tab_v,
              x_a, x_b, out_a, out_b,
              isem_a, isem_b, osem_a, osem_b):
    cid = lax.axis_index("c")
    sid = lax.axis_index("s")
    wid = sid * NC + cid
    eg = lax.rem(wid, NEG)
    ug = lax.div(wid, NEG)
    e0 = eg * EPG
    ebias = e0 * V

    pltpu.sync_copy(tab_hbm, tab_v)

    def x_copy(i, v, sem):
        off = (ug * UPG + i * SU) * UW
        return pltpu.make_async_copy(
            x_hbm.at[pl.ds(off, SW)], v, sem)

    def out_copies(i, v, sem):
        u0 = ug * UPG + i * SU
        return [pltpu.make_async_copy(
                    v.at[pl.ds(j * SW, SW)],
                    out_hbm.at[pl.ds((e0 + j) * P + u0 * UW, SW)], sem)
                for j in range(EPG)]

    def out_start(i, v, sem):
        for c in out_copies(i, v, sem):
            c.start()

    def out_wait(i, v, sem):
        for c in out_copies(i, v, sem):
            c.wait()

    def compute(xv, ov):
        def vec(vi, _):
            iv = xv[pl.ds(vi * 16, 16)] + ebias
            gs = [plsc.load_gather(tab_v, [iv + j * V])
                  for j in range(EPG)]
            for j in range(EPG):
                ov[pl.ds(j * SW + vi * 16, 16)] = gs[j]
            return 0
        lax.fori_loop(0, NV, vec, 0, unroll=4)

    bufs = ((x_a, out_a, isem_a, osem_a), (x_b, out_b, isem_b, osem_b))

    x_copy(0, x_a, isem_a).start()
    x_copy(1, x_b, isem_b).start()
    for i in (0, 1):
        xv, ov, isem, osem = bufs[i]
        x_copy(i, xv, isem).wait()
        compute(xv, ov)
        out_start(i, ov, osem)
        x_copy(i + 2, xv, isem).start()

    def steady(h, _):
        for b in range(2):
            i = 2 * h + b
            xv, ov, isem, osem = bufs[b]
            x_copy(i, xv, isem).wait()
            out_wait(i - 2, ov, osem)
            compute(xv, ov)
            out_start(i, ov, osem)
            x_copy(lax.rem(i + 2, NIT), xv, isem).start()
        return 0
    lax.fori_loop(1, NIT // 2, steady, 0)

    for b in range(2):
        xv, ov, isem, osem = bufs[b]
        out_wait(NIT - 2 + b, ov, osem)
        x_copy(b, xv, isem).wait()


@functools.partial(jax.jit, static_argnames=())
def kernel(input_x, table):
    xs = jnp.squeeze(input_x, 1).astype(jnp.int32)
    x_phys = xs.reshape(128, 128, 25, 8).transpose(2, 0, 3, 1).reshape(-1)
    tab_t = jnp.transpose(table).reshape(-1)

    run = pl.kernel(
        _emb_body,
        out_type=jax.ShapeDtypeStruct((E * P,), jnp.float32),
        mesh=plsc.VectorSubcoreMesh(
            core_axis_name="c", subcore_axis_name="s",
            num_cores=NC, num_subcores=NS),
        scratch_types=[
            pltpu.VMEM((E * V,), jnp.float32),
            pltpu.VMEM((SW,), jnp.int32),
            pltpu.VMEM((SW,), jnp.int32),
            pltpu.VMEM((EPG * SW,), jnp.float32),
            pltpu.VMEM((EPG * SW,), jnp.float32),
            pltpu.SemaphoreType.DMA,
            pltpu.SemaphoreType.DMA,
            pltpu.SemaphoreType.DMA,
            pltpu.SemaphoreType.DMA,
        ],
        compiler_params=pltpu.CompilerParams(needs_layout_passes=False),
    )
    out_phys = run(tab_t, x_phys)
    return (out_phys.reshape(E, 25, 128, 8, 128)
            .transpose(2, 4, 0, 1, 3).reshape(B, E, L))


# speedup vs baseline: 49.2676x; 1.2309x over previous
import functools

import jax
import jax.numpy as jnp
from jax import lax
from jax.experimental import pallas as pl
from jax.experimental.pallas import tpu as pltpu
from jax.experimental.pallas import tpu_sc as plsc

B = 16384
L = 200
E = 32
V = 257

NC = 2
NS = 16
NW = NC * NS
P = B * L
NEG = 4
NUG = NW // NEG
EPG = E // NEG
UW = 1024
UNITS = P // UW
UPG = UNITS // NUG
SU = 4
SW = SU * UW
NIT = UPG // SU
NV = SW // 16


def _emb_body(tab_hbm, x_hbm, out_hbm, tab_v,
              x_a, x_b, out_a, out_b,
              isem_a, isem_b, osem_a, osem_b):
    cid = lax.axis_index("c")
    sid = lax.axis_index("s")
    wid = sid * NC + cid
    eg = lax.rem(wid, NEG)
    ug = lax.div(wid, NEG)
    e0 = eg * EPG

    pltpu.sync_copy(tab_hbm.at[pl.ds(e0 * V * 16, EPG * V * 16)], tab_v)
    lanes = lax.broadcasted_iota(jnp.int32, (16,), 0)

    def x_copy(i, v, sem):
        off = (ug * UPG + i * SU) * UW
        return pltpu.make_async_copy(
            x_hbm.at[pl.ds(off, SW)], v, sem)

    def out_copies(i, v, sem):
        u0 = ug * UPG + i * SU
        return [pltpu.make_async_copy(
                    v.at[pl.ds(j * SW, SW)],
                    out_hbm.at[pl.ds((e0 + j) * P + u0 * UW, SW)], sem)
                for j in range(EPG)]

    def out_start(i, v, sem):
        for c in out_copies(i, v, sem):
            c.start()

    def out_wait(i, v, sem):
        for c in out_copies(i, v, sem):
            c.wait()

    def compute(xv, ov):
        def vec(vi, _):
            iv = xv[pl.ds(vi * 16, 16)] * 16 + lanes
            gs = [plsc.load_gather(tab_v, [iv + j * (V * 16)])
                  for j in range(EPG)]
            for j in range(EPG):
                ov[pl.ds(j * SW + vi * 16, 16)] = gs[j]
            return 0
        lax.fori_loop(0, NV, vec, 0, unroll=16)

    bufs = ((x_a, out_a, isem_a, osem_a), (x_b, out_b, isem_b, osem_b))

    x_copy(0, x_a, isem_a).start()
    x_copy(1, x_b, isem_b).start()
    for i in (0, 1):
        xv, ov, isem, osem = bufs[i]
        x_copy(i, xv, isem).wait()
        compute(xv, ov)
        out_start(i, ov, osem)
        x_copy(i + 2, xv, isem).start()

    def steady(h, _):
        for b in range(2):
            i = 2 * h + b
            xv, ov, isem, osem = bufs[b]
            x_copy(i, xv, isem).wait()
            out_wait(i - 2, ov, osem)
            compute(xv, ov)
            out_start(i, ov, osem)
            x_copy(lax.rem(i + 2, NIT), xv, isem).start()
        return 0
    lax.fori_loop(1, NIT // 2, steady, 0)

    for b in range(2):
        xv, ov, isem, osem = bufs[b]
        out_wait(NIT - 2 + b, ov, osem)
        x_copy(b, xv, isem).wait()


@functools.partial(jax.jit, static_argnames=())
def kernel(input_x, table):
    xs = jnp.squeeze(input_x, 1).astype(jnp.int32)
    x_phys = xs.reshape(128, 128, 25, 8).transpose(2, 0, 3, 1).reshape(-1)
    tab_t = jnp.broadcast_to(jnp.transpose(table).reshape(-1)[:, None],
                             (E * V, 16)).reshape(-1)

    run = pl.kernel(
        _emb_body,
        out_type=jax.ShapeDtypeStruct((E * P,), jnp.float32),
        mesh=plsc.VectorSubcoreMesh(
            core_axis_name="c", subcore_axis_name="s",
            num_cores=NC, num_subcores=NS),
        scratch_types=[
            pltpu.VMEM((EPG * V * 16,), jnp.float32),
            pltpu.VMEM((SW,), jnp.int32),
            pltpu.VMEM((SW,), jnp.int32),
            pltpu.VMEM((EPG * SW,), jnp.float32),
            pltpu.VMEM((EPG * SW,), jnp.float32),
            pltpu.SemaphoreType.DMA,
            pltpu.SemaphoreType.DMA,
            pltpu.SemaphoreType.DMA,
            pltpu.SemaphoreType.DMA,
        ],
        compiler_params=pltpu.CompilerParams(needs_layout_passes=False),
    )
    out_phys = run(tab_t, x_phys)
    return (out_phys.reshape(E, 25, 128, 8, 128)
            .transpose(2, 4, 0, 1, 3).reshape(B, E, L))
